# Initial kernel scaffold; baseline (speedup 1.0000x reference)
import jax
import jax.numpy as jnp
from jax.experimental import pallas as pl


def kernel(x, enc_table, ln_w, ln_b):
    raise NotImplementedError("write your pallas kernel here")


# speedup vs baseline: 1.7948x; 1.7948x over previous
import functools

import jax
import jax.numpy as jnp
from jax.experimental import pallas as pl
from jax.experimental.pallas import tpu as pltpu

_EPS = 1e-06


def _ln_body(x_ref, enc_ref, w_ref, b_ref, o_ref):
    y = x_ref[0] + enc_ref[...]
    mean = jnp.mean(y, axis=-1, keepdims=True)
    yc = y - mean
    var = jnp.mean(yc * yc, axis=-1, keepdims=True)
    o_ref[0] = yc * jax.lax.rsqrt(var + _EPS) * w_ref[...] + b_ref[...]


@functools.partial(jax.jit, static_argnames=("bs",))
def _tempo_enc(x, enc_table, ln_w, ln_b, bs=512):
    batch, seq, n_attr = x.shape
    enc = enc_table[:seq]
    w2 = ln_w.reshape(1, n_attr)
    b2 = ln_b.reshape(1, n_attr)
    grid = (seq // bs, batch)
    return pl.pallas_call(
        _ln_body,
        grid=grid,
        in_specs=[
            pl.BlockSpec((1, bs, n_attr), lambda s, b: (b, s, 0)),
            pl.BlockSpec((bs, n_attr), lambda s, b: (s, 0)),
            pl.BlockSpec((1, n_attr), lambda s, b: (0, 0)),
            pl.BlockSpec((1, n_attr), lambda s, b: (0, 0)),
        ],
        out_specs=pl.BlockSpec((1, bs, n_attr), lambda s, b: (b, s, 0)),
        out_shape=jax.ShapeDtypeStruct(x.shape, x.dtype),
        compiler_params=pltpu.CompilerParams(
            dimension_semantics=("arbitrary", "arbitrary"),
        ),
    )(x, enc, w2, b2)


def kernel(x, enc_table, ln_w, ln_b):
    return _tempo_enc(x, enc_table, ln_w, ln_b)
